# baseline (speedup 1.0000x reference)
import jax
import jax.numpy as jnp
from jax import lax
from jax.experimental import pallas as pl
from jax.experimental.pallas import tpu as pltpu
from jax.experimental.pallas import tpu_sc as plsc

_N = 16384
_D = 4
_L = 16
_NC, _NS = 2, 16
_NW = _NC * _NS
_FLAT = _N * _D
_CHUNK = _FLAT // _NW


def _body(w1_hbm, w2_hbm, out_hbm, w1_v, w2_v, out_v):
    wid = lax.axis_index("s") * _NC + lax.axis_index("c")
    pltpu.sync_copy(w1_hbm, w1_v)
    pltpu.sync_copy(w2_hbm, w2_v)
    p = jnp.sign(w1_v[...]) * jnp.sign(w2_v[...])
    for i in range(_CHUNK // _L):
        out_v[pl.ds(i * _L, _L)] = p
    pltpu.sync_copy(out_v, out_hbm.at[pl.ds(wid * _CHUNK, _CHUNK)])


@jax.jit
def _run(w1, w2):
    w1_16 = jnp.tile(w1.reshape(-1), _L // _D)
    w2_16 = jnp.tile(w2.reshape(-1), _L // _D)
    mesh = plsc.VectorSubcoreMesh(core_axis_name="c", subcore_axis_name="s")
    return pl.kernel(
        _body,
        out_type=jax.ShapeDtypeStruct((_FLAT,), jnp.float32),
        mesh=mesh,
        scratch_types=[
            pltpu.VMEM((_L,), jnp.float32),
            pltpu.VMEM((_L,), jnp.float32),
            pltpu.VMEM((_CHUNK,), jnp.float32),
        ],
    )(w1_16, w2_16)


def kernel(input, data1_weight, data2_weight):
    del input
    return _run(data1_weight, data2_weight).reshape(_N, _D)


# speedup vs baseline: 1.0156x; 1.0156x over previous
import jax
import jax.numpy as jnp
from jax import lax
from jax.experimental import pallas as pl
from jax.experimental.pallas import tpu as pltpu
from jax.experimental.pallas import tpu_sc as plsc

_N = 16384
_D = 4
_L = 16
_NC, _NS = 2, 16
_NW = _NC * _NS
_FLAT = _N * _D
_CHUNK = _FLAT // _NW


def _body(w_hbm, out_hbm, w_v, out_v):
    wid = lax.axis_index("s") * _NC + lax.axis_index("c")
    pltpu.sync_copy(w_hbm, w_v)
    p = jnp.sign(w_v[pl.ds(0, _L)]) * jnp.sign(w_v[pl.ds(_L, _L)])
    for i in range(_CHUNK // _L):
        out_v[pl.ds(i * _L, _L)] = p
    pltpu.sync_copy(out_v, out_hbm.at[pl.ds(wid * _CHUNK, _CHUNK)])


@jax.jit
def _run(w1, w2):
    w = jnp.concatenate(
        [jnp.tile(w1.reshape(-1), _L // _D), jnp.tile(w2.reshape(-1), _L // _D)]
    )
    mesh = plsc.VectorSubcoreMesh(core_axis_name="c", subcore_axis_name="s")
    return pl.kernel(
        _body,
        out_type=jax.ShapeDtypeStruct((_FLAT,), jnp.float32),
        mesh=mesh,
        scratch_types=[
            pltpu.VMEM((2 * _L,), jnp.float32),
            pltpu.VMEM((_CHUNK,), jnp.float32),
        ],
    )(w)


def kernel(input, data1_weight, data2_weight):
    del input
    return _run(data1_weight, data2_weight).reshape(_N, _D)


# speedup vs baseline: 1.0829x; 1.0663x over previous
import jax
import jax.numpy as jnp
from jax import lax
from jax.experimental import pallas as pl
from jax.experimental.pallas import tpu as pltpu
from jax.experimental.pallas import tpu_sc as plsc

_N = 16384
_D = 4
_L = 16
_NC, _NS = 2, 16
_NW = _NC * _NS
_FLAT = _N * _D
_CHUNK = _FLAT // _NW


_ROWS = _N // _NW


def _body(w_hbm, out_hbm, w_v, out_v):
    wid = lax.axis_index("s") * _NC + lax.axis_index("c")
    pltpu.sync_copy(w_hbm, w_v)
    p = jnp.sign(w_v[pl.ds(0, _L)]) * jnp.sign(w_v[pl.ds(_L, _L)])
    lanes = lax.iota(jnp.int32, _L)
    row4 = jnp.right_shift(lanes, 2)
    col = jnp.bitwise_and(lanes, _D - 1)
    for i in range(_ROWS // _D):
        plsc.store_scatter(out_v, [row4 + _D * i, col], p)
    pltpu.sync_copy(out_v, out_hbm.at[pl.ds(wid * _ROWS, _ROWS)])


@jax.jit
def _run(w1, w2):
    w = jnp.concatenate(
        [jnp.tile(w1.reshape(-1), _L // _D), jnp.tile(w2.reshape(-1), _L // _D)]
    )
    mesh = plsc.VectorSubcoreMesh(core_axis_name="c", subcore_axis_name="s")
    return pl.kernel(
        _body,
        out_type=jax.ShapeDtypeStruct((_N, _D), jnp.float32),
        mesh=mesh,
        compiler_params=pltpu.CompilerParams(needs_layout_passes=False),
        scratch_types=[
            pltpu.VMEM((2 * _L,), jnp.float32),
            pltpu.VMEM((_ROWS, _D), jnp.float32),
        ],
    )(w)


def kernel(input, data1_weight, data2_weight):
    del input
    return _run(data1_weight, data2_weight)


# speedup vs baseline: 1.2983x; 1.1989x over previous
import jax
import jax.numpy as jnp
from jax import lax
from jax.experimental import pallas as pl
from jax.experimental.pallas import tpu as pltpu
from jax.experimental.pallas import tpu_sc as plsc

_N = 16384
_D = 4
_L = 16
_NC = 2


def _sc_body(w_hbm, out_hbm, w_v, p_v):
    wid = lax.axis_index("s") * _NC + lax.axis_index("c")

    @pl.when(wid == 0)
    def _():
        pltpu.sync_copy(w_hbm, w_v)
        p_v[...] = jnp.sign(w_v[pl.ds(0, _L)]) * jnp.sign(w_v[pl.ds(_L, _L)])
        pltpu.sync_copy(p_v, out_hbm.at[0])


def _tc_body(p_ref, out_ref):
    row = p_ref[0, pl.ds(0, _D)].reshape(1, _D)
    out_ref[...] = jnp.broadcast_to(row, (_N, _D))


@jax.jit
def _run(w1, w2):
    w = jnp.concatenate(
        [jnp.tile(w1.reshape(-1), _L // _D), jnp.tile(w2.reshape(-1), _L // _D)]
    )
    mesh = plsc.VectorSubcoreMesh(core_axis_name="c", subcore_axis_name="s")
    p = pl.kernel(
        _sc_body,
        out_type=jax.ShapeDtypeStruct((1, _L), jnp.float32),
        mesh=mesh,
        compiler_params=pltpu.CompilerParams(needs_layout_passes=False),
        scratch_types=[
            pltpu.VMEM((2 * _L,), jnp.float32),
            pltpu.VMEM((_L,), jnp.float32),
        ],
    )(w)
    return pl.pallas_call(
        _tc_body,
        out_shape=jax.ShapeDtypeStruct((_N, _D), jnp.float32),
    )(p)


def kernel(input, data1_weight, data2_weight):
    del input
    return _run(data1_weight, data2_weight)
